# baseline (speedup 1.0000x reference)
import jax
import jax.numpy as jnp
from jax import lax
from jax.experimental import pallas as pl
from jax.experimental.pallas import tpu as pltpu
from jax.experimental.pallas import tpu_sc as plsc

N_USER = 100000
N_ITEM = 1000000
EMBED_DIM = 32
BATCH = 16384
PACK = 128 // EMBED_DIM

NC = 2
NS = 16
NW = NC * NS
R = BATCH // NW
CHUNK = 128
NCHUNK = R // CHUNK
HALF = R // 2
GROUPS = R // 16


def _bpr_body(us_hbm, is_hbm, js_hbm, uo_hbm, io_hbm, jo_hbm,
              eu_hbm, ei_hbm, out_ui_hbm, out_uj_hbm,
              usidx, isidx, jsidx, uoff, ioff, joff,
              ue, ie, je, oui, ouj, sem):
    wid = lax.axis_index("s") * NC + lax.axis_index("c")
    base = wid * R

    pltpu.sync_copy(us_hbm.at[wid], usidx)
    pltpu.sync_copy(is_hbm.at[wid], isidx)
    pltpu.sync_copy(js_hbm.at[wid], jsidx)
    pltpu.sync_copy(uo_hbm.at[wid], uoff)
    pltpu.sync_copy(io_hbm.at[wid], ioff)
    pltpu.sync_copy(jo_hbm.at[wid], joff)

    lane = lax.iota(jnp.int32, 16)

    for hp in range(2):
        copies = []
        for k in range(NCHUNK // 2):
            c = hp * (NCHUNK // 2) + k
            dst = pl.ds(k * CHUNK, CHUNK)
            copies.append(
                pltpu.async_copy(eu_hbm.at[usidx.at[c]], ue.at[dst], sem))
            copies.append(
                pltpu.async_copy(ei_hbm.at[isidx.at[c]], ie.at[dst], sem))
            copies.append(
                pltpu.async_copy(ei_hbm.at[jsidx.at[c]], je.at[dst], sem))
        for cp in copies:
            cp.wait()

        def group(gl, _):
            g = hp * (GROUPS // 2) + gl
            c = g // 8
            lo = (g % 8) * 16
            rowv = lane + gl * 16
            uov = uoff[c, pl.ds(lo, 16)]
            iov = ioff[c, pl.ds(lo, 16)]
            jov = joff[c, pl.ds(lo, 16)]
            acc_ui = jnp.zeros((16,), jnp.float32)
            acc_uj = jnp.zeros((16,), jnp.float32)
            for d in range(EMBED_DIM):
                uev = plsc.load_gather(ue, [rowv, uov + d])
                iev = plsc.load_gather(ie, [rowv, iov + d])
                jev = plsc.load_gather(je, [rowv, jov + d])
                acc_ui = acc_ui + uev * iev
                acc_uj = acc_uj + uev * jev
            oui[pl.ds(g * 16, 16)] = acc_ui
            ouj[pl.ds(g * 16, 16)] = acc_uj
            return _

        lax.fori_loop(0, GROUPS // 2, group, None)

    pltpu.sync_copy(oui, out_ui_hbm.at[pl.ds(base, R)])
    pltpu.sync_copy(ouj, out_uj_hbm.at[pl.ds(base, R)])


@jax.jit
def _bpr(us, is_, js, uo, io, jo, eu2, ei2):
    mesh = plsc.VectorSubcoreMesh(core_axis_name="c", subcore_axis_name="s")
    f = pl.kernel(
        _bpr_body,
        out_type=(
            jax.ShapeDtypeStruct((BATCH,), jnp.float32),
            jax.ShapeDtypeStruct((BATCH,), jnp.float32),
        ),
        mesh=mesh,
        compiler_params=pltpu.CompilerParams(needs_layout_passes=False),
        scratch_types=[
            pltpu.VMEM((NCHUNK, CHUNK), jnp.int32),
            pltpu.VMEM((NCHUNK, CHUNK), jnp.int32),
            pltpu.VMEM((NCHUNK, CHUNK), jnp.int32),
            pltpu.VMEM((NCHUNK, CHUNK), jnp.int32),
            pltpu.VMEM((NCHUNK, CHUNK), jnp.int32),
            pltpu.VMEM((NCHUNK, CHUNK), jnp.int32),
            pltpu.VMEM((HALF, 128), jnp.float32),
            pltpu.VMEM((HALF, 128), jnp.float32),
            pltpu.VMEM((HALF, 128), jnp.float32),
            pltpu.VMEM((R,), jnp.float32),
            pltpu.VMEM((R,), jnp.float32),
            pltpu.SemaphoreType.DMA,
        ],
    )
    return f(us, is_, js, uo, io, jo, eu2, ei2)


def kernel(u, i, j, embed_user, embed_item):
    u1 = u.astype(jnp.int32).reshape(-1)
    i1 = i.astype(jnp.int32).reshape(-1)
    j1 = j.astype(jnp.int32).reshape(-1)
    shape3 = (NW, NCHUNK, CHUNK)
    us = (u1 // PACK).reshape(shape3)
    is_ = (i1 // PACK).reshape(shape3)
    js = (j1 // PACK).reshape(shape3)
    uo = ((u1 % PACK) * EMBED_DIM).reshape(shape3)
    io = ((i1 % PACK) * EMBED_DIM).reshape(shape3)
    jo = ((j1 % PACK) * EMBED_DIM).reshape(shape3)
    eu2 = embed_user.reshape(N_USER // PACK, 128)
    ei2 = embed_item.reshape(N_ITEM // PACK, 128)
    p_ui, p_uj = _bpr(us, is_, js, uo, io, jo, eu2, ei2)
    return (p_ui.reshape(BATCH, 1), p_uj.reshape(BATCH, 1))


# speedup vs baseline: 1.0009x; 1.0009x over previous
import jax
import jax.numpy as jnp
from jax import lax
from jax.experimental import pallas as pl
from jax.experimental.pallas import tpu as pltpu
from jax.experimental.pallas import tpu_sc as plsc

N_USER = 100000
N_ITEM = 1000000
EMBED_DIM = 32
BATCH = 16384
PACK = 128 // EMBED_DIM

NC = 2
NS = 16
NW = NC * NS
R = BATCH // NW
CHUNK = 128
NCHUNK = R // CHUNK
HALF = R // 2
GROUPS = R // 16


def _bpr_body(u_hbm, i_hbm, j_hbm, eu_hbm, ei_hbm, out_ui_hbm, out_uj_hbm,
              uraw, iraw, jraw, usidx, isidx, jsidx, uoff, ioff, joff,
              ue, ie, je, oui, ouj, sem):
    wid = lax.axis_index("s") * NC + lax.axis_index("c")

    pltpu.sync_copy(u_hbm.at[wid], uraw)
    pltpu.sync_copy(i_hbm.at[wid], iraw)
    pltpu.sync_copy(j_hbm.at[wid], jraw)

    for c in range(NCHUNK):
        for g in range(CHUNK // 16):
            s = pl.ds(g * 16, 16)
            uv = uraw[c, s]
            iv = iraw[c, s]
            jv = jraw[c, s]
            usidx[c, s] = uv // PACK
            isidx[c, s] = iv // PACK
            jsidx[c, s] = jv // PACK
            uoff[c, s] = (uv % PACK) * EMBED_DIM
            ioff[c, s] = (iv % PACK) * EMBED_DIM
            joff[c, s] = (jv % PACK) * EMBED_DIM

    lane = lax.iota(jnp.int32, 16)
    base = wid * R

    for hp in range(2):
        copies = []
        for k in range(NCHUNK // 2):
            c = hp * (NCHUNK // 2) + k
            dst = pl.ds(k * CHUNK, CHUNK)
            copies.append(
                pltpu.async_copy(eu_hbm.at[usidx.at[c]], ue.at[dst], sem))
            copies.append(
                pltpu.async_copy(ei_hbm.at[isidx.at[c]], ie.at[dst], sem))
            copies.append(
                pltpu.async_copy(ei_hbm.at[jsidx.at[c]], je.at[dst], sem))
        for cp in copies:
            cp.wait()

        def group(gl, _):
            g = hp * (GROUPS // 2) + gl
            c = g // 8
            lo = (g % 8) * 16
            rowv = lane + gl * 16
            uov = uoff[c, pl.ds(lo, 16)]
            iov = ioff[c, pl.ds(lo, 16)]
            jov = joff[c, pl.ds(lo, 16)]
            acc_ui = jnp.zeros((16,), jnp.float32)
            acc_uj = jnp.zeros((16,), jnp.float32)
            for d in range(EMBED_DIM):
                uev = plsc.load_gather(ue, [rowv, uov + d])
                iev = plsc.load_gather(ie, [rowv, iov + d])
                jev = plsc.load_gather(je, [rowv, jov + d])
                acc_ui = acc_ui + uev * iev
                acc_uj = acc_uj + uev * jev
            oui[pl.ds(g * 16, 16)] = acc_ui
            ouj[pl.ds(g * 16, 16)] = acc_uj
            return _

        lax.fori_loop(0, GROUPS // 2, group, None)

    pltpu.sync_copy(oui, out_ui_hbm.at[pl.ds(base, R)])
    pltpu.sync_copy(ouj, out_uj_hbm.at[pl.ds(base, R)])


@jax.jit
def _bpr(u3, i3, j3, eu2, ei2):
    mesh = plsc.VectorSubcoreMesh(core_axis_name="c", subcore_axis_name="s")
    f = pl.kernel(
        _bpr_body,
        out_type=(
            jax.ShapeDtypeStruct((BATCH,), jnp.float32),
            jax.ShapeDtypeStruct((BATCH,), jnp.float32),
        ),
        mesh=mesh,
        compiler_params=pltpu.CompilerParams(needs_layout_passes=False),
        scratch_types=[
            pltpu.VMEM((NCHUNK, CHUNK), jnp.int32),
            pltpu.VMEM((NCHUNK, CHUNK), jnp.int32),
            pltpu.VMEM((NCHUNK, CHUNK), jnp.int32),
            pltpu.VMEM((NCHUNK, CHUNK), jnp.int32),
            pltpu.VMEM((NCHUNK, CHUNK), jnp.int32),
            pltpu.VMEM((NCHUNK, CHUNK), jnp.int32),
            pltpu.VMEM((NCHUNK, CHUNK), jnp.int32),
            pltpu.VMEM((NCHUNK, CHUNK), jnp.int32),
            pltpu.VMEM((NCHUNK, CHUNK), jnp.int32),
            pltpu.VMEM((HALF, 128), jnp.float32),
            pltpu.VMEM((HALF, 128), jnp.float32),
            pltpu.VMEM((HALF, 128), jnp.float32),
            pltpu.VMEM((R,), jnp.float32),
            pltpu.VMEM((R,), jnp.float32),
            pltpu.SemaphoreType.DMA,
        ],
    )
    return f(u3, i3, j3, eu2, ei2)


def kernel(u, i, j, embed_user, embed_item):
    shape3 = (NW, NCHUNK, CHUNK)
    u3 = u.astype(jnp.int32).reshape(shape3)
    i3 = i.astype(jnp.int32).reshape(shape3)
    j3 = j.astype(jnp.int32).reshape(shape3)
    eu2 = embed_user.reshape(N_USER // PACK, 128)
    ei2 = embed_item.reshape(N_ITEM // PACK, 128)
    p_ui, p_uj = _bpr(u3, i3, j3, eu2, ei2)
    return (p_ui.reshape(BATCH, 1), p_uj.reshape(BATCH, 1))
